# baseline (speedup 1.0000x reference)
import functools

import jax
import jax.numpy as jnp
from jax import lax
from jax.experimental import pallas as pl
from jax.experimental.pallas import tpu as pltpu
from jax.experimental.pallas import tpu_sc as plsc

NC = 2
NS = 16
L = 16
B = 80


def _bcast_lane(v, j):
  idx = jnp.full((L, 1), j, jnp.int32)
  dnums = lax.GatherDimensionNumbers(
      offset_dims=(), collapsed_slice_dims=(0,), start_index_map=(0,))
  return lax.gather(v, idx, dnums, slice_sizes=(1,),
                    mode=lax.GatherScatterMode.PROMISE_IN_BOUNDS)


def _sc_scatter(x, src, dst, attr):
  N, D = x.shape
  E = src.shape[0]
  NW = NC * NS
  EPT = E // NW
  NCH = EPT // B
  RPT = (N // NS) // 8 * 8
  TAIL = N - RPT * NS
  ZR = 16
  EP = 2 + (NCH - 2) % 3
  assert E % (NW * B) == 0 and NCH >= EP + 3 and RPT % ZR == 0
  assert TAIL % 8 == 0 and TAIL <= ZR and D % L == 0

  mesh = plsc.VectorSubcoreMesh(
      core_axis_name="c", subcore_axis_name="s", num_cores=NC,
      num_subcores=NS)

  @functools.partial(
      pl.kernel,
      out_type=jax.ShapeDtypeStruct((NC, N, D), jnp.float32),
      mesh=mesh,
      scratch_types=[
          pltpu.VMEM_SHARED((N, D), jnp.float32),
          pltpu.VMEM((EPT,), jnp.int32),
          [pltpu.VMEM((B,), jnp.int32)] * 3,
          [pltpu.VMEM((B,), jnp.float32)] * 3,
          [pltpu.VMEM((B, D), jnp.float32)] * 3,
          pltpu.VMEM((ZR, D), jnp.float32),
          [pltpu.SemaphoreType.DMA] * 3,
          [pltpu.SemaphoreType.DMA] * 3,
          [pltpu.SemaphoreType.DMA] * 3,
          [pltpu.SemaphoreType.DMA] * 3,
      ],
  )
  def k(x_hbm, src_hbm, dst_hbm, attr_hbm, parts_hbm,
        acc_sh, src_all, dstb, attrb, rows_v, zbuf, gsem, wsem, dsem, asem):
    c = lax.axis_index("c")
    s = lax.axis_index("s")
    wid = c * NS + s
    abase = wid * EPT

    def issue_gather(i, b):
      off = pl.ds(pl.multiple_of(i * B, B), B)
      goff = pl.ds(abase + i * B, B)
      pltpu.async_copy(x_hbm.at[src_all.at[off]], rows_v[b], gsem[b])
      pltpu.async_copy(dst_hbm.at[goff], dstb[b], dsem[b])
      pltpu.async_copy(attr_hbm.at[goff], attrb[b], asem[b])

    def wait_gather(i, b):
      off = pl.ds(pl.multiple_of(i * B, B), B)
      goff = pl.ds(abase + i * B, B)
      pltpu.make_async_copy(x_hbm.at[src_all.at[off]], rows_v[b],
                            gsem[b]).wait()
      pltpu.make_async_copy(dst_hbm.at[goff], dstb[b], dsem[b]).wait()
      pltpu.make_async_copy(attr_hbm.at[goff], attrb[b], asem[b]).wait()

    def issue_scatter(i, b):
      pltpu.async_copy(rows_v[b], acc_sh.at[dstb[b].at[pl.ds(0, B)]],
                       wsem[b], add=True)

    def wait_scatter(b):
      pltpu.make_async_copy(rows_v[b], acc_sh.at[dstb[b].at[pl.ds(0, B)]],
                            wsem[b]).wait()

    def scale(i, b):
      for g in range(B // L):
        r0 = g * L
        av = attrb[b][pl.ds(r0, L)]
        for j in range(L):
          a = _bcast_lane(av, j)
          for q in range(D // L):
            sl = pl.ds(q * L, L)
            rows_v[b][r0 + j, sl] = rows_v[b][r0 + j, sl] * a

    pltpu.sync_copy(src_hbm.at[pl.ds(abase, EPT)], src_all)
    issue_gather(0, 0)
    issue_gather(1, 1)

    def zrow(j, carry):
      for q in range(D // L):
        zbuf[j, pl.ds(q * L, L)] = jnp.zeros((L,), jnp.float32)
      return carry
    lax.fori_loop(0, ZR, zrow, 0)
    for t in range(RPT // ZR):
      pltpu.sync_copy(zbuf, acc_sh.at[pl.ds(s * RPT + t * ZR, ZR)])

    @pl.when(s == 0)
    def _():
      pltpu.sync_copy(zbuf.at[pl.ds(0, TAIL)],
                      acc_sh.at[pl.ds(NS * RPT, TAIL)])
    plsc.subcore_barrier()

    def step(i, b, in_loop):
      wait_gather(i, b)
      scale(i, b)
      if in_loop:
        @pl.when(i >= 1)
        def _():
          wait_scatter((b + 2) % 3)
        issue_gather(i + 2, (b + 2) % 3)
      issue_scatter(i, b)

    def triple(t, carry):
      i = t * 3
      step(i, 0, True)
      step(i + 1, 1, True)
      step(i + 2, 2, True)
      return carry
    lax.fori_loop(0, (NCH - EP) // 3, triple, 0)
    for k in range(EP):
      i = NCH - EP + k
      step(i, i % 3, k <= EP - 3)
    wait_scatter((NCH - 3) % 3)
    wait_scatter((NCH - 2) % 3)
    wait_scatter((NCH - 1) % 3)
    plsc.subcore_barrier()

    pltpu.sync_copy(acc_sh.at[pl.ds(s * RPT, RPT)],
                    parts_hbm.at[c, pl.ds(s * RPT, RPT)])

    @pl.when(s == 0)
    def _():
      pltpu.sync_copy(acc_sh.at[pl.ds(NS * RPT, TAIL)],
                      parts_hbm.at[c, pl.ds(NS * RPT, TAIL)])

  return k(x, src, dst, attr)


def _tc_root_body(x_ref, wroot_ref, b_ref, out_ref):
  dn = (((1,), (1,)), ((), ()))
  out_ref[...] = (
      lax.dot_general(x_ref[...], wroot_ref[...], dn,
                      preferred_element_type=jnp.float32,
                      precision=lax.Precision.HIGHEST)
      + b_ref[...]
  )


def _tc_root(x, W_root, b_rel):
  N, D = x.shape
  BM = 400
  return pl.pallas_call(
      _tc_root_body,
      grid=(N // BM,),
      in_specs=[
          pl.BlockSpec((BM, D), lambda i: (i, 0)),
          pl.BlockSpec((D, D), lambda i: (0, 0)),
          pl.BlockSpec((1, D), lambda i: (0, 0)),
      ],
      out_specs=pl.BlockSpec((BM, D), lambda i: (i, 0)),
      out_shape=jax.ShapeDtypeStruct((N, D), jnp.float32),
  )(x, W_root, b_rel.reshape(1, D))


def _tc_rel_body(parts_ref, root_ref, wrel_ref, out_ref):
  agg = parts_ref[0] + parts_ref[1]
  dn = (((1,), (1,)), ((), ()))
  out_ref[...] = (
      lax.dot_general(agg, wrel_ref[...], dn,
                      preferred_element_type=jnp.float32,
                      precision=lax.Precision.HIGHEST)
      + root_ref[...]
  )


def _tc_rel(parts, root, W_rel):
  N, D = root.shape
  BM = 400
  return pl.pallas_call(
      _tc_rel_body,
      grid=(N // BM,),
      in_specs=[
          pl.BlockSpec((NC, BM, D), lambda i: (0, i, 0)),
          pl.BlockSpec((BM, D), lambda i: (i, 0)),
          pl.BlockSpec((D, D), lambda i: (0, 0)),
      ],
      out_specs=pl.BlockSpec((BM, D), lambda i: (i, 0)),
      out_shape=jax.ShapeDtypeStruct((N, D), jnp.float32),
  )(parts, root, W_rel)


@jax.jit
def kernel(x, edge_index, edge_attr, W_rel, b_rel, W_root):
  src = edge_index[0]
  dst = edge_index[1]
  parts = _sc_scatter(x, src, dst, edge_attr)
  root = _tc_root(x, W_root, b_rel)
  return _tc_rel(parts, root, W_rel)


# speedup vs baseline: 1.2517x; 1.2517x over previous
import functools

import jax
import jax.numpy as jnp
from jax import lax
from jax.experimental import pallas as pl
from jax.experimental.pallas import tpu as pltpu
from jax.experimental.pallas import tpu_sc as plsc

NC = 2
NS = 16
L = 16
B = 80


def _bcast_lane(v, j):
  idx = jnp.full((L, 1), j, jnp.int32)
  dnums = lax.GatherDimensionNumbers(
      offset_dims=(), collapsed_slice_dims=(0,), start_index_map=(0,))
  return lax.gather(v, idx, dnums, slice_sizes=(1,),
                    mode=lax.GatherScatterMode.PROMISE_IN_BOUNDS)


def _sc_scatter(x, src, dst, attr):
  N, D = x.shape
  E = src.shape[0]
  NW = NC * NS
  EPT = E // NW
  NCH = EPT // B
  RPT = (N // NS) // 8 * 8
  TAIL = N - RPT * NS
  ZR = 16
  EP = 2 + (NCH - 2) % 3
  assert E % (NW * B) == 0 and NCH >= EP + 3 and RPT % ZR == 0
  assert TAIL % 8 == 0 and TAIL <= ZR and D % L == 0

  mesh = plsc.VectorSubcoreMesh(
      core_axis_name="c", subcore_axis_name="s", num_cores=NC,
      num_subcores=NS)

  @functools.partial(
      pl.kernel,
      out_type=jax.ShapeDtypeStruct((NC, N, D), jnp.float32),
      mesh=mesh,
      scratch_types=[
          pltpu.VMEM_SHARED((N, D), jnp.float32),
          pltpu.VMEM((EPT,), jnp.int32),
          [pltpu.VMEM((B,), jnp.int32)] * 3,
          [pltpu.VMEM((B,), jnp.float32)] * 3,
          [pltpu.VMEM((B, D), jnp.float32)] * 3,
          pltpu.VMEM((ZR, D), jnp.float32),
          [pltpu.SemaphoreType.DMA] * 3,
          [pltpu.SemaphoreType.DMA] * 3,
          [pltpu.SemaphoreType.DMA] * 3,
          [pltpu.SemaphoreType.DMA] * 3,
      ],
  )
  def k(x_hbm, src_hbm, dst_hbm, attr_hbm, parts_hbm,
        acc_sh, src_all, dstb, attrb, rows_v, zbuf, gsem, wsem, dsem, asem):
    c = lax.axis_index("c")
    s = lax.axis_index("s")
    wid = c * NS + s
    abase = wid * EPT

    def issue_gather(i, b):
      off = pl.ds(pl.multiple_of(i * B, B), B)
      goff = pl.ds(abase + i * B, B)
      pltpu.async_copy(x_hbm.at[src_all.at[off]], rows_v[b], gsem[b])
      pltpu.async_copy(dst_hbm.at[goff], dstb[b], dsem[b])
      pltpu.async_copy(attr_hbm.at[goff], attrb[b], asem[b])

    def wait_gather(i, b):
      off = pl.ds(pl.multiple_of(i * B, B), B)
      goff = pl.ds(abase + i * B, B)
      pltpu.make_async_copy(x_hbm.at[src_all.at[off]], rows_v[b],
                            gsem[b]).wait()
      pltpu.make_async_copy(dst_hbm.at[goff], dstb[b], dsem[b]).wait()
      pltpu.make_async_copy(attr_hbm.at[goff], attrb[b], asem[b]).wait()

    def issue_scatter(i, b):
      pltpu.async_copy(rows_v[b], acc_sh.at[dstb[b].at[pl.ds(0, B)]],
                       wsem[b], add=True)

    def wait_scatter(b):
      pltpu.make_async_copy(rows_v[b], acc_sh.at[dstb[b].at[pl.ds(0, B)]],
                            wsem[b]).wait()

    def scale(i, b):
      def group(g, gcarry):
        r0 = pl.multiple_of(g * L, L)
        av = attrb[b][pl.ds(r0, L)]
        for j in range(L):
          a = _bcast_lane(av, j)
          for q in range(D // L):
            sl = pl.ds(q * L, L)
            rows_v[b][r0 + j, sl] = rows_v[b][r0 + j, sl] * a
        return gcarry
      lax.fori_loop(0, B // L, group, 0)

    pltpu.sync_copy(src_hbm.at[pl.ds(abase, EPT)], src_all)
    issue_gather(0, 0)
    issue_gather(1, 1)

    def zrow(j, carry):
      for q in range(D // L):
        zbuf[j, pl.ds(q * L, L)] = jnp.zeros((L,), jnp.float32)
      return carry
    lax.fori_loop(0, ZR, zrow, 0)
    for t in range(RPT // ZR):
      pltpu.sync_copy(zbuf, acc_sh.at[pl.ds(s * RPT + t * ZR, ZR)])

    @pl.when(s == 0)
    def _():
      pltpu.sync_copy(zbuf.at[pl.ds(0, TAIL)],
                      acc_sh.at[pl.ds(NS * RPT, TAIL)])
    plsc.subcore_barrier()

    def step(i, b, in_loop):
      wait_gather(i, b)
      scale(i, b)
      if in_loop:
        @pl.when(i >= 1)
        def _():
          wait_scatter((b + 2) % 3)
        issue_gather(i + 2, (b + 2) % 3)
      issue_scatter(i, b)

    def triple(t, carry):
      i = t * 3
      step(i, 0, True)
      step(i + 1, 1, True)
      step(i + 2, 2, True)
      return carry
    lax.fori_loop(0, (NCH - EP) // 3, triple, 0)
    for k in range(EP):
      i = NCH - EP + k
      step(i, i % 3, k <= EP - 3)
    wait_scatter((NCH - 3) % 3)
    wait_scatter((NCH - 2) % 3)
    wait_scatter((NCH - 1) % 3)
    plsc.subcore_barrier()

    pltpu.sync_copy(acc_sh.at[pl.ds(s * RPT, RPT)],
                    parts_hbm.at[c, pl.ds(s * RPT, RPT)])

    @pl.when(s == 0)
    def _():
      pltpu.sync_copy(acc_sh.at[pl.ds(NS * RPT, TAIL)],
                      parts_hbm.at[c, pl.ds(NS * RPT, TAIL)])

  return k(x, src, dst, attr)


def _tc_root_body(x_ref, wroot_ref, b_ref, out_ref):
  dn = (((1,), (1,)), ((), ()))
  out_ref[...] = (
      lax.dot_general(x_ref[...], wroot_ref[...], dn,
                      preferred_element_type=jnp.float32,
                      precision=lax.Precision.HIGHEST)
      + b_ref[...]
  )


def _tc_root(x, W_root, b_rel):
  N, D = x.shape
  BM = 400
  return pl.pallas_call(
      _tc_root_body,
      grid=(N // BM,),
      in_specs=[
          pl.BlockSpec((BM, D), lambda i: (i, 0)),
          pl.BlockSpec((D, D), lambda i: (0, 0)),
          pl.BlockSpec((1, D), lambda i: (0, 0)),
      ],
      out_specs=pl.BlockSpec((BM, D), lambda i: (i, 0)),
      out_shape=jax.ShapeDtypeStruct((N, D), jnp.float32),
  )(x, W_root, b_rel.reshape(1, D))


def _tc_rel_body(parts_ref, root_ref, wrel_ref, out_ref):
  agg = parts_ref[0] + parts_ref[1]
  dn = (((1,), (1,)), ((), ()))
  out_ref[...] = (
      lax.dot_general(agg, wrel_ref[...], dn,
                      preferred_element_type=jnp.float32,
                      precision=lax.Precision.HIGHEST)
      + root_ref[...]
  )


def _tc_rel(parts, root, W_rel):
  N, D = root.shape
  BM = 400
  return pl.pallas_call(
      _tc_rel_body,
      grid=(N // BM,),
      in_specs=[
          pl.BlockSpec((NC, BM, D), lambda i: (0, i, 0)),
          pl.BlockSpec((BM, D), lambda i: (i, 0)),
          pl.BlockSpec((D, D), lambda i: (0, 0)),
      ],
      out_specs=pl.BlockSpec((BM, D), lambda i: (i, 0)),
      out_shape=jax.ShapeDtypeStruct((N, D), jnp.float32),
  )(parts, root, W_rel)


@jax.jit
def kernel(x, edge_index, edge_attr, W_rel, b_rel, W_root):
  src = edge_index[0]
  dst = edge_index[1]
  parts = _sc_scatter(x, src, dst, edge_attr)
  root = _tc_root(x, W_root, b_rel)
  return _tc_rel(parts, root, W_rel)
